# baseline (speedup 1.0000x reference)
import functools

import jax
import jax.numpy as jnp
from jax import lax
from jax.experimental import pallas as pl
from jax.experimental.pallas import tpu as pltpu

RATE = 1.0
B, C_IN, H, W = 8, 192, 224, 224
C_OUT = 192
K = int(C_OUT * RATE)
SPATIAL = H * W
COL_BLOCK = 1792
NSTEPS = SPATIAL // COL_BLOCK


def _reduce_gate_kernel(x_ref, w_ref, b_ref, g_ref, s_acc):
    j = pl.program_id(0)
    part = jnp.sum(jnp.abs(x_ref[...]), axis=2)

    @pl.when(j == 0)
    def _init():
        s_acc[...] = part

    @pl.when(j > 0)
    def _acc():
        s_acc[...] = s_acc[...] + part

    @pl.when(j == NSTEPS - 1)
    def _gate():
        s = s_acc[...] * (1.0 / SPATIAL)
        g = lax.dot_general(s, w_ref[...], (((1,), (1,)), ((), ())),
                            preferred_element_type=jnp.float32)
        g_ref[...] = jnp.maximum(g + b_ref[...], 0.0)


def _mask_kernel(g_ref, t_ref):
    g = g_ref[...]
    ge = g[:, :, None]
    gm = g[:, None, :]
    e_idx = lax.broadcasted_iota(jnp.int32, (B, C_OUT, C_OUT), 1)
    m_idx = lax.broadcasted_iota(jnp.int32, (B, C_OUT, C_OUT), 2)
    smaller = (gm < ge) | ((gm == ge) & (m_idx < e_idx))
    rank = jnp.sum(smaller.astype(jnp.int32), axis=2)
    t = jnp.where(rank >= K, g, 0.0)
    t_sum = jnp.sum(t, axis=1, keepdims=True)
    t_ref[...] = t / t_sum * C_OUT


@jax.jit
def kernel(x, gate_w, gate_b):
    x3 = x.reshape(B, C_IN, SPATIAL)
    w_main = gate_w[:, :C_IN]
    b_eff = (gate_b + RATE * gate_w[:, C_IN]).reshape(1, C_OUT)

    g = pl.pallas_call(
        _reduce_gate_kernel,
        grid=(NSTEPS,),
        in_specs=[
            pl.BlockSpec((B, C_IN, COL_BLOCK), lambda j: (0, 0, j)),
            pl.BlockSpec((C_OUT, C_IN), lambda j: (0, 0)),
            pl.BlockSpec((1, C_OUT), lambda j: (0, 0)),
        ],
        out_specs=pl.BlockSpec((B, C_OUT), lambda j: (0, 0)),
        out_shape=jax.ShapeDtypeStruct((B, C_OUT), jnp.float32),
        scratch_shapes=[pltpu.VMEM((B, C_IN), jnp.float32)],
    )(x3, w_main, b_eff)

    t = pl.pallas_call(
        _mask_kernel,
        out_shape=jax.ShapeDtypeStruct((B, C_OUT), jnp.float32),
    )(g)
    return t[:, :, None, None]
